# baseline (speedup 1.0000x reference)
import functools

import jax
import jax.numpy as jnp
from jax import lax
from jax.experimental import pallas as pl
from jax.experimental.pallas import tpu as pltpu
from jax.experimental.pallas import tpu_sc as plsc

N = 10000
E = 320000
D = 128
NC = 2
NS = 16
CH = 96
EPC = E // NC
PER_SUB = 104
EPS = PER_SUB * CH
HALF = PER_SUB // 2
HCH = HALF * CH
NB = 3
REM_E = EPC - NS * EPS
REM_CH = 64
REM_W = REM_E // REM_CH
RPS = 624
TAIL = N - NS * RPS


def _sc_agg_body(h_ref, e_ref, z_ref, acc, src_v, dst_v, rows_v,
                 gsem, ssem):
    src_ref = e_ref.at[0]
    dst_ref = e_ref.at[1]
    c = lax.axis_index("c")
    s = lax.axis_index("s")
    r0 = s * RPS
    e0 = c * EPC + s * EPS

    def issue_gather(g, b):
        pltpu.async_copy(
            h_ref.at[src_v.at[pl.ds(g * CH, CH)]], rows_v.at[b], gsem.at[b])

    def drain_gather(g, b):
        pltpu.make_async_copy(
            h_ref.at[src_v.at[pl.ds(g * CH, CH)]], rows_v.at[b], gsem.at[b]).wait()

    def issue_scatter(g, b):
        pltpu.async_copy(
            rows_v.at[b], acc.at[dst_v.at[pl.ds(g * CH, CH)]], ssem.at[b], add=True)

    def drain_scatter(g, b):
        pltpu.make_async_copy(
            rows_v.at[b], acc.at[dst_v.at[pl.ds(g * CH, CH)]], ssem.at[b]).wait()

    pltpu.sync_copy(h_ref.at[pl.ds(r0, RPS)], acc.at[pl.ds(r0, RPS)])

    @pl.when(s == 0)
    def _():
        pltpu.sync_copy(h_ref.at[pl.ds(NS * RPS, TAIL)], acc.at[pl.ds(NS * RPS, TAIL)])

    plsc.subcore_barrier()

    def run_half(h):
        base = e0 + h * HCH
        pltpu.sync_copy(src_ref.at[pl.ds(base, HCH)], src_v)
        pltpu.sync_copy(dst_ref.at[pl.ds(base, HCH)], dst_v)
        issue_gather(0, 0)
        issue_gather(1, 1)

        @pl.loop(0, (HALF - 4) // NB)
        def _(i):
            for j in range(NB):
                g = 2 + 3 * i + j
                b = (2 + j) % NB
                if j == 0:
                    @pl.when(i > 0)
                    def _():
                        drain_scatter(g - 3, b)
                else:
                    drain_scatter(g - 3, b)
                issue_gather(g, b)
                drain_gather(g - 2, j)
                issue_scatter(g - 2, j)

        for g in (50, 51):
            drain_scatter(g - 3, g % NB)
            issue_gather(g, g % NB)
            drain_gather(g - 2, (g - 2) % NB)
            issue_scatter(g - 2, (g - 2) % NB)
        for g in (50, 51):
            drain_gather(g, g % NB)
            issue_scatter(g, g % NB)
        for g in (49, 50, 51):
            drain_scatter(g, g % NB)

    run_half(0)
    run_half(1)

    @pl.when(s < REM_W)
    def _():
        base = c * EPC + NS * EPS + s * REM_CH
        pltpu.sync_copy(src_ref.at[pl.ds(base, REM_CH)], src_v.at[pl.ds(0, REM_CH)])
        pltpu.sync_copy(dst_ref.at[pl.ds(base, REM_CH)], dst_v.at[pl.ds(0, REM_CH)])
        pltpu.sync_copy(h_ref.at[src_v.at[pl.ds(0, REM_CH)]],
                        rows_v.at[0, pl.ds(0, REM_CH)])
        pltpu.sync_copy(rows_v.at[0, pl.ds(0, REM_CH)],
                        acc.at[dst_v.at[pl.ds(0, REM_CH)]], add=True)

    plsc.subcore_barrier()
    pltpu.sync_copy(acc.at[pl.ds(r0, RPS)], z_ref.at[c, pl.ds(r0, RPS)])

    @pl.when(s == 0)
    def _():
        pltpu.sync_copy(acc.at[pl.ds(NS * RPS, TAIL)], z_ref.at[c, pl.ds(NS * RPS, TAIL)])


@functools.cache
def _get_sc_agg():
    mesh = plsc.VectorSubcoreMesh(
        core_axis_name="c", subcore_axis_name="s", num_cores=NC, num_subcores=NS)
    return pl.kernel(
        _sc_agg_body,
        out_type=jax.ShapeDtypeStruct((NC, N, D), jnp.float32),
        mesh=mesh,
        scratch_types=[
            pltpu.VMEM_SHARED((N, D), jnp.float32),
            pltpu.VMEM((HCH,), jnp.int32),
            pltpu.VMEM((HCH,), jnp.int32),
            pltpu.VMEM((NB, CH, D), jnp.float32),
            pltpu.SemaphoreType.DMA((NB,)),
            pltpu.SemaphoreType.DMA((NB,)),
        ],
    )


def _dense_body(relu_out, z_ref, h_ref, w1_ref, b1_ref, g1_ref, be1_ref,
                w2_ref, b2_ref, go_ref, bo_ref, out_ref):
    z = z_ref[0] + z_ref[1] - h_ref[...]
    t = jnp.dot(z.astype(jnp.bfloat16), w1_ref[...].astype(jnp.bfloat16),
                preferred_element_type=jnp.float32) + b1_ref[...]
    mean = jnp.mean(t, axis=0)
    var = jnp.mean((t - mean) ** 2, axis=0)
    t = (t - mean) / jnp.sqrt(var + 1e-5) * g1_ref[...] + be1_ref[...]
    t = jnp.maximum(t, 0.0)
    u = jnp.dot(t.astype(jnp.bfloat16), w2_ref[...].astype(jnp.bfloat16),
                preferred_element_type=jnp.float32) + b2_ref[...]
    mean2 = jnp.mean(u, axis=0)
    var2 = jnp.mean((u - mean2) ** 2, axis=0)
    u = (u - mean2) / jnp.sqrt(var2 + 1e-5) * go_ref[...] + bo_ref[...]
    if relu_out:
        u = jnp.maximum(u, 0.0)
    out_ref[...] = u


def _make_dense(relu_out):
    return pl.pallas_call(
        functools.partial(_dense_body, relu_out),
        out_shape=jax.ShapeDtypeStruct((N, D), jnp.float32),
    )


_dense_mid = _make_dense(relu_out=True)
_dense_last = _make_dense(relu_out=False)


def kernel(x, edge_index, edge_attr, batch,
           W1_0, b1_0, g1_0, be1_0, W2_0, b2_0, go_0, bo_0,
           W1_1, b1_1, g1_1, be1_1, W2_1, b2_1, go_1, bo_1):
    sc_agg = _get_sc_agg()
    p = sc_agg(x, edge_index)
    h = _dense_mid(p, x, W1_0, b1_0, g1_0, be1_0, W2_0, b2_0, go_0, bo_0)
    p = sc_agg(h, edge_index)
    return _dense_last(p, h, W1_1, b1_1, g1_1, be1_1, W2_1, b2_1, go_1, bo_1)


# speedup vs baseline: 1.0226x; 1.0226x over previous
import functools

import jax
import jax.numpy as jnp
from jax import lax
from jax.experimental import pallas as pl
from jax.experimental.pallas import tpu as pltpu
from jax.experimental.pallas import tpu_sc as plsc

N = 10000
E = 320000
D = 128
NC = 2
NS = 16
CH = 96
EPC = E // NC
PER_SUB = 104
EPS = PER_SUB * CH
HALF = PER_SUB // 2
HCH = HALF * CH
NB = 3
REM_E = EPC - NS * EPS
REM_CH = 64
REM_W = REM_E // REM_CH
RPS = 624
TAIL = N - NS * RPS


def _sc_agg_body(h_ref, e_ref, z_ref, acc, src_v, dst_v, rows_v,
                 gsem, ssem):
    src_ref = e_ref.at[0]
    dst_ref = e_ref.at[1]
    c = lax.axis_index("c")
    s = lax.axis_index("s")
    r0 = s * RPS
    e0 = c * EPC + s * EPS

    def issue_gather(g, b):
        pltpu.async_copy(
            h_ref.at[src_v.at[pl.ds(g * CH, CH)]], rows_v.at[b], gsem.at[b])

    def drain_gather(g, b):
        pltpu.make_async_copy(
            h_ref.at[src_v.at[pl.ds(g * CH, CH)]], rows_v.at[b], gsem.at[b]).wait()

    def issue_scatter(g, b):
        pltpu.async_copy(
            rows_v.at[b], acc.at[dst_v.at[pl.ds(g * CH, CH)]], ssem.at[b], add=True)

    def drain_scatter(g, b):
        pltpu.make_async_copy(
            rows_v.at[b], acc.at[dst_v.at[pl.ds(g * CH, CH)]], ssem.at[b]).wait()

    pltpu.sync_copy(h_ref.at[pl.ds(r0, RPS)], acc.at[pl.ds(r0, RPS)])

    @pl.when(s == 0)
    def _():
        pltpu.sync_copy(h_ref.at[pl.ds(NS * RPS, TAIL)], acc.at[pl.ds(NS * RPS, TAIL)])

    plsc.subcore_barrier()

    def run_half(h):
        base = e0 + h * HCH
        pltpu.sync_copy(src_ref.at[pl.ds(base, HCH)], src_v)
        pltpu.sync_copy(dst_ref.at[pl.ds(base, HCH)], dst_v)
        issue_gather(0, 0)
        issue_gather(1, 1)

        @pl.loop(0, (HALF - 4) // NB)
        def _(i):
            for j in range(NB):
                g = 2 + 3 * i + j
                b = (2 + j) % NB
                if j == 0:
                    @pl.when(i > 0)
                    def _():
                        drain_scatter(g - 3, b)
                else:
                    drain_scatter(g - 3, b)
                issue_gather(g, b)
                drain_gather(g - 2, j)
                issue_scatter(g - 2, j)

        for g in (50, 51):
            drain_scatter(g - 3, g % NB)
            issue_gather(g, g % NB)
            drain_gather(g - 2, (g - 2) % NB)
            issue_scatter(g - 2, (g - 2) % NB)
        for g in (50, 51):
            drain_gather(g, g % NB)
            issue_scatter(g, g % NB)
        for g in (49, 50, 51):
            drain_scatter(g, g % NB)

    run_half(0)
    run_half(1)

    @pl.when(s < REM_W)
    def _():
        base = c * EPC + NS * EPS + s * REM_CH
        pltpu.sync_copy(src_ref.at[pl.ds(base, REM_CH)], src_v.at[pl.ds(0, REM_CH)])
        pltpu.sync_copy(dst_ref.at[pl.ds(base, REM_CH)], dst_v.at[pl.ds(0, REM_CH)])
        pltpu.sync_copy(h_ref.at[src_v.at[pl.ds(0, REM_CH)]],
                        rows_v.at[0, pl.ds(0, REM_CH)])
        pltpu.sync_copy(rows_v.at[0, pl.ds(0, REM_CH)],
                        acc.at[dst_v.at[pl.ds(0, REM_CH)]], add=True)

    plsc.subcore_barrier()
    pltpu.sync_copy(acc.at[pl.ds(r0, RPS)], z_ref.at[c, pl.ds(r0, RPS)])

    @pl.when(s == 0)
    def _():
        pltpu.sync_copy(acc.at[pl.ds(NS * RPS, TAIL)], z_ref.at[c, pl.ds(NS * RPS, TAIL)])


@functools.cache
def _get_sc_agg():
    mesh = plsc.VectorSubcoreMesh(
        core_axis_name="c", subcore_axis_name="s", num_cores=NC, num_subcores=NS)
    return pl.kernel(
        _sc_agg_body,
        out_type=jax.ShapeDtypeStruct((NC, N, D), jnp.float32),
        mesh=mesh,
        scratch_types=[
            pltpu.VMEM_SHARED((N, D), jnp.float32),
            pltpu.VMEM((HCH,), jnp.int32),
            pltpu.VMEM((HCH,), jnp.int32),
            pltpu.VMEM((NB, CH, D), jnp.float32),
            pltpu.SemaphoreType.DMA((NB,)),
            pltpu.SemaphoreType.DMA((NB,)),
        ],
    )


def _dense_body(relu_out, z_ref, h_ref, w1_ref, b1_ref, g1_ref, be1_ref,
                w2_ref, b2_ref, go_ref, bo_ref, out_ref):
    z = z_ref[0] + z_ref[1] - h_ref[...]
    t = jnp.dot(z, w1_ref[...], preferred_element_type=jnp.float32) + b1_ref[...]
    mean = jnp.mean(t, axis=0)
    var = jnp.mean((t - mean) ** 2, axis=0)
    t = (t - mean) / jnp.sqrt(var + 1e-5) * g1_ref[...] + be1_ref[...]
    t = jnp.maximum(t, 0.0)
    u = jnp.dot(t, w2_ref[...], preferred_element_type=jnp.float32) + b2_ref[...]
    mean2 = jnp.mean(u, axis=0)
    var2 = jnp.mean((u - mean2) ** 2, axis=0)
    u = (u - mean2) / jnp.sqrt(var2 + 1e-5) * go_ref[...] + bo_ref[...]
    if relu_out:
        u = jnp.maximum(u, 0.0)
    out_ref[...] = u


def _make_dense(relu_out):
    return pl.pallas_call(
        functools.partial(_dense_body, relu_out),
        out_shape=jax.ShapeDtypeStruct((N, D), jnp.float32),
    )


_dense_mid = _make_dense(relu_out=True)
_dense_last = _make_dense(relu_out=False)


def kernel(x, edge_index, edge_attr, batch,
           W1_0, b1_0, g1_0, be1_0, W2_0, b2_0, go_0, bo_0,
           W1_1, b1_1, g1_1, be1_1, W2_1, b2_1, go_1, bo_1):
    sc_agg = _get_sc_agg()
    p = sc_agg(x, edge_index)
    h = _dense_mid(p, x, W1_0, b1_0, g1_0, be1_0, W2_0, b2_0, go_0, bo_0)
    p = sc_agg(h, edge_index)
    return _dense_last(p, h, W1_1, b1_1, g1_1, be1_1, W2_1, b2_1, go_1, bo_1)
